# baseline (speedup 1.0000x reference)
import functools
import jax
import jax.numpy as jnp
from jax import lax
from jax.experimental import pallas as pl
from jax.experimental.pallas import tpu as pltpu, tpu_sc as plsc

N, D, E, R, H = 2048, 1024, 8, 16, 256
BN = 512
GRID = N // BN
NW = 32
TPW = N // NW
NG = TPW // 16

_mesh = plsc.VectorSubcoreMesh(core_axis_name="c", subcore_axis_name="s")


def _gate_body(x_ref, wg1_ref, bg1_ref, wg2_ref, bg2_ref, lg_ref):
    xb = x_ref[...]
    h = jax.lax.dot_general(xb, wg1_ref[...], (((1,), (1,)), ((), ())),
                            preferred_element_type=jnp.float32)
    h = jnp.maximum(h + bg1_ref[...], 0.0)
    logits = jax.lax.dot_general(wg2_ref[...], h, (((1,), (1,)), ((), ())),
                                 preferred_element_type=jnp.float32)
    lg_ref[...] = logits + bg2_ref[...]


@functools.partial(
    pl.kernel, mesh=_mesh,
    out_type=[
        jax.ShapeDtypeStruct((E * N,), jnp.float32),
        jax.ShapeDtypeStruct((NW * E * 16,), jnp.float32),
        jax.ShapeDtypeStruct((NW * E * 16,), jnp.float32),
    ],
    scratch_types=[
        pltpu.VMEM((E, TPW), jnp.float32),
        pltpu.VMEM((E, TPW), jnp.float32),
        pltpu.VMEM((E * 16,), jnp.float32),
        pltpu.VMEM((E * 16,), jnp.float32),
    ],
)
def _sc_route(logits_hbm, wt_hbm, pm_hbm, pp_hbm, lg_v, wt_v, pm_v, pp_v):
    wid = lax.axis_index("s") * 2 + lax.axis_index("c")
    base = wid * TPW
    for e in range(E):
        pltpu.sync_copy(logits_hbm.at[pl.ds(e * N + base, TPW)], lg_v.at[e])
    accm = [jnp.zeros((16,), jnp.float32) for _ in range(E)]
    accp = [jnp.zeros((16,), jnp.float32) for _ in range(E)]
    for g in range(NG):
        v = [lg_v[e, pl.ds(g * 16, 16)] for e in range(E)]
        m1 = v[0]
        i1 = jnp.zeros((16,), jnp.int32)
        for e in range(1, E):
            take = v[e] > m1
            m1 = jnp.where(take, v[e], m1)
            i1 = jnp.where(take, e, i1)
        m2 = jnp.full((16,), -jnp.inf, jnp.float32)
        i2 = jnp.zeros((16,), jnp.int32)
        for e in range(E):
            take = (v[e] > m2) & (i1 != e)
            m2 = jnp.where(take, v[e], m2)
            i2 = jnp.where(take, e, i2)
        a = jnp.exp(m2 - m1)
        g1 = 1.0 / (1.0 + a)
        g2 = 1.0 - g1
        ex = [jnp.exp(v[e] - m1) for e in range(E)]
        s = ex[0]
        for e in range(1, E):
            s = s + ex[e]
        inv = 1.0 / s
        for e in range(E):
            sel1 = i1 == e
            sel2 = i2 == e
            we = jnp.where(sel1, g1, 0.0) + jnp.where(sel2, g2, 0.0)
            wt_v[e, pl.ds(g * 16, 16)] = we
            accm[e] = (accm[e] + jnp.where(sel1, 1.0, 0.0)
                       + jnp.where(sel2, 1.0, 0.0))
            accp[e] = accp[e] + ex[e] * inv
    for e in range(E):
        pm_v[pl.ds(e * 16, 16)] = accm[e]
        pp_v[pl.ds(e * 16, 16)] = accp[e]
    for e in range(E):
        pltpu.sync_copy(wt_v.at[e], wt_hbm.at[pl.ds(e * N + base, TPW)])
    pltpu.sync_copy(pm_v, pm_hbm.at[pl.ds(wid * E * 16, E * 16)])
    pltpu.sync_copy(pp_v, pp_hbm.at[pl.ds(wid * E * 16, E * 16)])


def _main_body(x_ref, wf_ref, a_ref, b_ref, bias_ref, wt_ref, pm_ref, pp_ref,
               out_ref, loss_ref):
    step = pl.program_id(0)
    xb = x_ref[...]
    of = jax.lax.dot_general(xb, wf_ref[...], (((1,), (1,)), ((), ())),
                             preferred_element_type=jnp.float32)
    xa = jax.lax.dot_general(xb, a_ref[...], (((1,), (1,)), ((), ())),
                             preferred_element_type=jnp.float32)

    rows = jax.lax.broadcasted_iota(jnp.int32, (E, E * R), 0)
    cols = jax.lax.broadcasted_iota(jnp.int32, (E, E * R), 1)
    sel = (cols // R == rows).astype(jnp.float32)
    w_wide = jax.lax.dot_general(wt_ref[...], sel, (((0,), (0,)), ((), ())),
                                 preferred_element_type=jnp.float32)

    eo = jax.lax.dot_general(xa * w_wide, b_ref[...], (((1,), (0,)), ((), ())),
                             preferred_element_type=jnp.float32)
    out_ref[...] = of + eo + bias_ref[...]

    @pl.when(step == GRID - 1)
    def _():
        am = pm_ref[0:E, :]
        ap = pp_ref[0:E, :]
        for wkr in range(1, NW):
            am = am + pm_ref[wkr * E:(wkr + 1) * E, :]
            ap = ap + pp_ref[wkr * E:(wkr + 1) * E, :]
        am1 = jnp.sum(am, axis=1, keepdims=True)
        ap1 = jnp.sum(ap, axis=1, keepdims=True)
        loss = (E / (N * N)) * jnp.sum(am1 * ap1, keepdims=True)
        loss_ref[...] = loss.reshape(1, 1)


@jax.jit
def _run(x, Wg1, bg1, Wg2, bg2, weight_fixed, A_flat, B_flat, bias):
    full = lambda s: pl.BlockSpec(s, lambda i: (0, 0))
    logitsT = pl.pallas_call(
        _gate_body,
        grid=(GRID,),
        in_specs=[
            pl.BlockSpec((BN, D), lambda i: (i, 0)),
            full((H, D)),
            full((1, H)),
            full((E, H)),
            full((E, 1)),
        ],
        out_specs=pl.BlockSpec((E, BN), lambda i: (0, i)),
        out_shape=jax.ShapeDtypeStruct((E, N), jnp.float32),
        compiler_params=pltpu.CompilerParams(
            dimension_semantics=("arbitrary",)),
    )(x, Wg1, bg1.reshape(1, H), Wg2, bg2.reshape(E, 1))

    wt_f, pm_f, pp_f = _sc_route(logitsT.reshape(E * N))
    wt = wt_f.reshape(E, N)
    pm = pm_f.reshape(NW * E, 16)
    pp = pp_f.reshape(NW * E, 16)

    out, loss = pl.pallas_call(
        _main_body,
        grid=(GRID,),
        in_specs=[
            pl.BlockSpec((BN, D), lambda i: (i, 0)),
            full((D, D)),
            full((E * R, D)),
            full((E * R, D)),
            full((1, D)),
            pl.BlockSpec((E, BN), lambda i: (0, i)),
            full((NW * E, 16)),
            full((NW * E, 16)),
        ],
        out_specs=[
            pl.BlockSpec((BN, D), lambda i: (i, 0)),
            full((1, 1)),
        ],
        out_shape=[
            jax.ShapeDtypeStruct((N, D), jnp.float32),
            jax.ShapeDtypeStruct((1, 1), jnp.float32),
        ],
        compiler_params=pltpu.CompilerParams(
            dimension_semantics=("arbitrary",)),
    )(x, weight_fixed, A_flat, B_flat, bias.reshape(1, D), wt, pm, pp)
    return out, loss[0, 0]


def kernel(x, Wg1, bg1, Wg2, bg2, weight_fixed, A_all, B_all, bias):
    A_flat = A_all.reshape(E * R, D)
    B_flat = B_all.transpose(0, 2, 1).reshape(E * R, D)
    return _run(x, Wg1, bg1, Wg2, bg2, weight_fixed, A_flat, B_flat, bias)


# speedup vs baseline: 2.6164x; 2.6164x over previous
import jax
import jax.numpy as jnp
from jax.experimental import pallas as pl
from jax.experimental.pallas import tpu as pltpu

N = 2048
D = 1024
E = 8
R = 16
H = 256
K = 2

BN = 512
GRID = N // BN


def _body(x_ref, wg1_ref, bg1_ref, wg2_ref, bg2_ref, wf_ref, a_ref, b_ref,
          bias_ref, out_ref, loss_ref, acc_m, acc_p):
    step = pl.program_id(0)
    xb = x_ref[...]

    h = jax.lax.dot_general(xb, wg1_ref[...], (((1,), (1,)), ((), ())),
                            preferred_element_type=jnp.float32)
    h = jnp.maximum(h + bg1_ref[...], 0.0)
    logits = jax.lax.dot_general(wg2_ref[...], h, (((1,), (1,)), ((), ())),
                                 preferred_element_type=jnp.float32)
    logits = logits + bg2_ref[...]

    neg = jnp.float32(-jnp.inf)
    m1 = jnp.full((1, BN), neg, jnp.float32)
    i1 = jnp.zeros((1, BN), jnp.int32)
    for e in range(E):
        v = logits[e:e + 1, :]
        take = v > m1
        m1 = jnp.where(take, v, m1)
        i1 = jnp.where(take, e, i1)
    m2 = jnp.full((1, BN), neg, jnp.float32)
    i2 = jnp.zeros((1, BN), jnp.int32)
    for e in range(E):
        v = logits[e:e + 1, :]
        take = (v > m2) & (i1 != e)
        m2 = jnp.where(take, v, m2)
        i2 = jnp.where(take, e, i2)

    a = jnp.exp(m2 - m1)
    g1 = 1.0 / (1.0 + a)
    g2 = 1.0 - g1

    eiota = jax.lax.broadcasted_iota(jnp.int32, (E, BN), 0)
    sel1 = eiota == i1
    sel2 = eiota == i2
    wt = jnp.where(sel1, g1, 0.0) + jnp.where(sel2, g2, 0.0)
    maskt = (sel1 | sel2).astype(jnp.float32)

    ex = jnp.exp(logits - m1)
    probst = ex / jnp.sum(ex, axis=0, keepdims=True)

    of = jax.lax.dot_general(xb, wf_ref[...], (((1,), (1,)), ((), ())),
                             preferred_element_type=jnp.float32)
    xa = jax.lax.dot_general(xb, a_ref[...], (((1,), (1,)), ((), ())),
                             preferred_element_type=jnp.float32)

    rows = jax.lax.broadcasted_iota(jnp.int32, (E, E * R), 0)
    cols = jax.lax.broadcasted_iota(jnp.int32, (E, E * R), 1)
    sel = (cols // R == rows).astype(jnp.float32)
    w_wide = jax.lax.dot_general(wt, sel, (((0,), (0,)), ((), ())),
                                 preferred_element_type=jnp.float32)

    eo = jax.lax.dot_general(xa * w_wide, b_ref[...], (((1,), (0,)), ((), ())),
                             preferred_element_type=jnp.float32)
    out_ref[...] = of + eo + bias_ref[...]

    @pl.when(step == 0)
    def _():
        acc_m[...] = maskt
        acc_p[...] = probst

    @pl.when(step > 0)
    def _():
        acc_m[...] += maskt
        acc_p[...] += probst

    @pl.when(step == GRID - 1)
    def _():
        am = jnp.sum(acc_m[...], axis=1, keepdims=True)
        ap = jnp.sum(acc_p[...], axis=1, keepdims=True)
        loss = (E / (N * N)) * jnp.sum(am * ap, keepdims=True)
        loss_ref[...] = loss.reshape(1, 1)


@jax.jit
def _run(x, Wg1, bg1, Wg2, bg2, weight_fixed, A_flat, B_flat, bias):
    full = lambda s: pl.BlockSpec(s, lambda i: (0, 0))
    out, loss = pl.pallas_call(
        _body,
        grid=(GRID,),
        in_specs=[
            pl.BlockSpec((BN, D), lambda i: (i, 0)),
            full((H, D)),
            full((1, H)),
            full((E, H)),
            full((E, 1)),
            full((D, D)),
            full((E * R, D)),
            full((E * R, D)),
            full((1, D)),
        ],
        out_specs=[
            pl.BlockSpec((BN, D), lambda i: (i, 0)),
            full((1, 1)),
        ],
        out_shape=[
            jax.ShapeDtypeStruct((N, D), jnp.float32),
            jax.ShapeDtypeStruct((1, 1), jnp.float32),
        ],
        scratch_shapes=[
            pltpu.VMEM((E, BN), jnp.float32),
            pltpu.VMEM((E, BN), jnp.float32),
        ],
        compiler_params=pltpu.CompilerParams(
            dimension_semantics=("arbitrary",),
        ),
    )(x, Wg1, bg1.reshape(1, H), Wg2, bg2.reshape(E, 1),
      weight_fixed, A_flat, B_flat, bias.reshape(1, D))
    return out, loss[0, 0]


def kernel(x, Wg1, bg1, Wg2, bg2, weight_fixed, A_all, B_all, bias):
    A_flat = A_all.reshape(E * R, D)
    B_flat = B_all.transpose(0, 2, 1).reshape(E * R, D)
    return _run(x, Wg1, bg1, Wg2, bg2, weight_fixed, A_flat, B_flat, bias)
